# initial kernel scaffold (unmeasured)
import jax
import jax.numpy as jnp
from jax import lax
from jax.experimental import pallas as pl
from jax.experimental.pallas import tpu as pltpu

N_DEV = 4
SQ = 1024
SKV_LOC = 1024
HQ = 8
DH = 128
D = HQ * DH
STATS = 128
W = D + STATS
SCALE = 0.08838834764831843
BLK = 64


def kernel(x, Wq, K_ext, V_ext, Wo):
    def body(x_ref, wq_ref, k_ref, v_ref, wo_ref, out_ref,
             gather_ref, ctx_ref, send_sems, recv_sems):
        my = lax.axis_index("i")
        left = lax.rem(my + N_DEV - 1, N_DEV)
        right = lax.rem(my + 1, N_DEV)

        q = jnp.dot(x_ref[0], wq_ref[...], preferred_element_type=jnp.float32)

        rows = lax.broadcasted_iota(jnp.int32, (SQ, SKV_LOC), 0)
        cols = lax.broadcasted_iota(jnp.int32, (SQ, SKV_LOC), 1)
        qb = rows // BLK
        kb = my * (SKV_LOC // BLK) + cols // BLK
        mask = (qb == kb) | (kb == 0) | (((qb + kb) % 3) == 0)

        kfull = k_ref[0]
        vfull = v_ref[0]
        for h in range(HQ):
            qh = q[:, h * DH:(h + 1) * DH]
            kh = kfull[:, h, :]
            vh = vfull[:, h, :]
            s = lax.dot_general(qh, kh, (((1,), (1,)), ((), ())),
                                preferred_element_type=jnp.float32)
            w = jnp.where(mask, jnp.exp(s * SCALE), 0.0)
            lh = jnp.sum(w, axis=1, keepdims=True)
            ctx = jnp.dot(w, vh, preferred_element_type=jnp.float32)
            gather_ref[0, :, h * DH:(h + 1) * DH] = ctx
            gather_ref[0, :, D + h:D + h + 1] = lh

        barrier = pltpu.get_barrier_semaphore()
        for nbr in (left, right):
            pl.semaphore_signal(barrier, inc=1, device_id=(nbr,),
                                device_id_type=pl.DeviceIdType.MESH)
        pl.semaphore_wait(barrier, 2)

        for h in range(N_DEV - 1):
            src_slot = (N_DEV - h) % N_DEV
            dst_slot = N_DEV - 1 - h
            rdma = pltpu.make_async_remote_copy(
                src_ref=gather_ref.at[src_slot],
                dst_ref=gather_ref.at[dst_slot],
                send_sem=send_sems.at[h],
                recv_sem=recv_sems.at[h],
                device_id=(right,),
                device_id_type=pl.DeviceIdType.MESH,
            )
            rdma.start()
            rdma.wait()

        for h in range(HQ):
            c = (gather_ref[0, :, h * DH:(h + 1) * DH]
                 + gather_ref[1, :, h * DH:(h + 1) * DH]
                 + gather_ref[2, :, h * DH:(h + 1) * DH]
                 + gather_ref[3, :, h * DH:(h + 1) * DH])
            l = (gather_ref[0, :, D + h:D + h + 1]
                 + gather_ref[1, :, D + h:D + h + 1]
                 + gather_ref[2, :, D + h:D + h + 1]
                 + gather_ref[3, :, D + h:D + h + 1])
            ctx_ref[:, h * DH:(h + 1) * DH] = c / l

        out_ref[0] = jnp.dot(ctx_ref[...], wo_ref[...],
                             preferred_element_type=jnp.float32)

    return pl.pallas_call(
        body,
        out_shape=jax.ShapeDtypeStruct((1, SQ, D), jnp.float32),
        in_specs=[pl.BlockSpec(memory_space=pltpu.VMEM)] * 5,
        out_specs=pl.BlockSpec(memory_space=pltpu.VMEM),
        scratch_shapes=[
            pltpu.VMEM((N_DEV, SQ, W), jnp.float32),
            pltpu.VMEM((SQ, D), jnp.float32),
            pltpu.SemaphoreType.DMA((N_DEV - 1,)),
            pltpu.SemaphoreType.DMA((N_DEV - 1,)),
        ],
        compiler_params=pltpu.CompilerParams(collective_id=0),
    )(x, Wq, K_ext, V_ext, Wo)


# baseline (device time: 196714 ns/iter reference)
import jax
import jax.numpy as jnp
from jax import lax
from jax.experimental import pallas as pl
from jax.experimental.pallas import tpu as pltpu

N_DEV = 4
SQ = 1024
SKV_LOC = 1024
HQ = 8
DH = 128
D = HQ * DH
STATS = 128
W = D + STATS
SCALE = 0.08838834764831843
BLK = 64


def kernel(x, Wq, K_ext, V_ext, Wo):
    def body(x_ref, wq_ref, k_ref, v_ref, wo_ref, out_ref,
             gather_ref, ctx_ref, send_sems, recv_sems):
        my = lax.axis_index("i")
        left = lax.rem(my + N_DEV - 1, N_DEV)
        right = lax.rem(my + 1, N_DEV)

        q = jnp.dot(x_ref[0], wq_ref[...], preferred_element_type=jnp.float32)

        rows = lax.broadcasted_iota(jnp.int32, (SQ, SKV_LOC), 0)
        cols = lax.broadcasted_iota(jnp.int32, (SQ, SKV_LOC), 1)
        qb = rows // BLK
        kb = my * (SKV_LOC // BLK) + cols // BLK
        mask = (qb == kb) | (kb == 0) | (((qb + kb) % 3) == 0)

        kfull = k_ref[0]
        vfull = v_ref[0]
        for h in range(HQ):
            qh = q[:, h * DH:(h + 1) * DH]
            kh = kfull[:, h, :]
            vh = vfull[:, h, :]
            s = lax.dot_general(qh, kh, (((1,), (1,)), ((), ())),
                                preferred_element_type=jnp.float32)
            w = jnp.where(mask, jnp.exp(s * SCALE), 0.0)
            lh = jnp.sum(w, axis=1, keepdims=True)
            ctx = jnp.dot(w, vh, preferred_element_type=jnp.float32)
            gather_ref[0, :, h * DH:(h + 1) * DH] = ctx
            gather_ref[0, :, D + h:D + h + 1] = lh

        barrier = pltpu.get_barrier_semaphore()
        for nbr in (left, right):
            pl.semaphore_signal(barrier, inc=1, device_id=(nbr,),
                                device_id_type=pl.DeviceIdType.MESH)
        pl.semaphore_wait(barrier, 2)

        for h in range(N_DEV - 1):
            src_slot = (N_DEV - h) % N_DEV
            dst_slot = N_DEV - 1 - h
            rdma = pltpu.make_async_remote_copy(
                src_ref=gather_ref.at[src_slot],
                dst_ref=gather_ref.at[dst_slot],
                send_sem=send_sems.at[h],
                recv_sem=recv_sems.at[h],
                device_id=(right,),
                device_id_type=pl.DeviceIdType.MESH,
            )
            rdma.start()
            rdma.wait()

        for h in range(HQ):
            c = (gather_ref[0, :, h * DH:(h + 1) * DH]
                 + gather_ref[1, :, h * DH:(h + 1) * DH]
                 + gather_ref[2, :, h * DH:(h + 1) * DH]
                 + gather_ref[3, :, h * DH:(h + 1) * DH])
            l = (gather_ref[0, :, D + h:D + h + 1]
                 + gather_ref[1, :, D + h:D + h + 1]
                 + gather_ref[2, :, D + h:D + h + 1]
                 + gather_ref[3, :, D + h:D + h + 1])
            ctx_ref[:, h * DH:(h + 1) * DH] = c / l

        out_ref[0] = jnp.dot(ctx_ref[...], wo_ref[...],
                             preferred_element_type=jnp.float32)

    return pl.pallas_call(
        body,
        out_shape=jax.ShapeDtypeStruct((1, SQ, D), jnp.float32),
        in_specs=[pl.BlockSpec(memory_space=pltpu.VMEM)] * 5,
        out_specs=pl.BlockSpec(memory_space=pltpu.VMEM),
        scratch_shapes=[
            pltpu.VMEM((N_DEV, SQ, W), jnp.float32),
            pltpu.VMEM((SQ, D), jnp.float32),
            pltpu.SemaphoreType.DMA((N_DEV - 1,)),
            pltpu.SemaphoreType.DMA((N_DEV - 1,)),
        ],
        compiler_params=pltpu.CompilerParams(
            collective_id=0, vmem_limit_bytes=128 * 1024 * 1024
        ),
    )(x, Wq, K_ext, V_ext, Wo)


# device time: 84382 ns/iter; 2.3312x vs baseline; 2.3312x over previous
import jax
import jax.numpy as jnp
from jax import lax
from jax.experimental import pallas as pl
from jax.experimental.pallas import tpu as pltpu

N_DEV = 4
SQ = 1024
SKV_LOC = 1024
HQ = 8
DH = 128
D = HQ * DH
STATS = 128
W = D + STATS
CHUNK = SQ // N_DEV
SCALE = 0.08838834764831843
BLK = 64
BF = jnp.bfloat16
F32 = jnp.float32


def kernel(x, Wq, K_ext, V_ext, Wo):
    def body(x_ref, wq_ref, k_ref, v_ref, wo_ref, out_ref,
             part_ref, comb_ref, recv1_ref, recv2_ref, ag_ref,
             send_sems, recv_sems):
        my = lax.axis_index("i")
        p_x = 3 - my
        p_y = jnp.bitwise_xor(my, 1)

        q = jnp.dot(x_ref[0].astype(BF), wq_ref[...].astype(BF),
                    preferred_element_type=F32)

        rows = lax.broadcasted_iota(jnp.int32, (SQ, SKV_LOC), 0)
        cols = lax.broadcasted_iota(jnp.int32, (SQ, SKV_LOC), 1)
        qb = rows // BLK
        kb = my * (SKV_LOC // BLK) + cols // BLK
        mask = (qb == kb) | (kb == 0) | (((qb + kb) % 3) == 0)

        q16 = q.astype(BF)
        kfull = k_ref[0]
        vfull = v_ref[0]
        for h in range(HQ):
            kh = kfull[:, h, :].astype(BF)
            vh = vfull[:, h, :].astype(BF)
            s = lax.dot_general(q16[:, h * DH:(h + 1) * DH], kh,
                                (((1,), (1,)), ((), ())),
                                preferred_element_type=F32)
            w = jnp.where(mask, jnp.exp(s * SCALE), 0.0)
            lh = jnp.sum(w, axis=1, keepdims=True)
            ctx = jnp.dot(w.astype(BF), vh, preferred_element_type=F32)
            part_ref[:, h * DH:(h + 1) * DH] = ctx.astype(BF)
            part_ref[:, D + h:D + h + 1] = lh.astype(BF)

        barrier = pltpu.get_barrier_semaphore()
        for nbr in (p_x, p_y):
            pl.semaphore_signal(barrier, inc=1, device_id=(nbr,),
                                device_id_type=pl.DeviceIdType.MESH)
        pl.semaphore_wait(barrier, 2)

        send1 = jnp.where(my < 2, SQ // 2, 0)
        keep1 = (SQ // 2) - send1
        rs1 = pltpu.make_async_remote_copy(
            src_ref=part_ref.at[pl.ds(send1, SQ // 2)],
            dst_ref=recv1_ref,
            send_sem=send_sems.at[0], recv_sem=recv_sems.at[0],
            device_id=(p_x,), device_id_type=pl.DeviceIdType.MESH,
        )
        rs1.start()
        rs1.wait()
        comb = (part_ref[pl.ds(keep1, SQ // 2), :].astype(F32)
                + recv1_ref[...].astype(F32))
        comb_ref[...] = comb.astype(BF)

        keep2 = lax.rem(my, 2) * CHUNK
        send2 = CHUNK - keep2
        rs2 = pltpu.make_async_remote_copy(
            src_ref=comb_ref.at[pl.ds(send2, CHUNK)],
            dst_ref=recv2_ref,
            send_sem=send_sems.at[1], recv_sem=recv_sems.at[1],
            device_id=(p_y,), device_id_type=pl.DeviceIdType.MESH,
        )
        rs2.start()
        rs2.wait()
        final = (comb_ref[pl.ds(keep2, CHUNK), :].astype(F32)
                 + recv2_ref[...].astype(F32))

        outc = jnp.zeros((CHUNK, D), dtype=F32)
        for h in range(HQ):
            ctxn = final[:, h * DH:(h + 1) * DH] / final[:, D + h:D + h + 1]
            outc = outc + jnp.dot(
                ctxn.astype(BF), wo_ref[h * DH:(h + 1) * DH, :].astype(BF),
                preferred_element_type=F32)
        ag_ref[pl.ds(my * CHUNK, CHUNK), :] = outc.astype(BF)

        aga = pltpu.make_async_remote_copy(
            src_ref=ag_ref.at[pl.ds(my * CHUNK, CHUNK)],
            dst_ref=ag_ref.at[pl.ds(my * CHUNK, CHUNK)],
            send_sem=send_sems.at[2], recv_sem=recv_sems.at[2],
            device_id=(p_y,), device_id_type=pl.DeviceIdType.MESH,
        )
        aga.start()
        aga.wait()
        half = (my // 2) * (SQ // 2)
        agb = pltpu.make_async_remote_copy(
            src_ref=ag_ref.at[pl.ds(half, SQ // 2)],
            dst_ref=ag_ref.at[pl.ds(half, SQ // 2)],
            send_sem=send_sems.at[3], recv_sem=recv_sems.at[3],
            device_id=(p_x,), device_id_type=pl.DeviceIdType.MESH,
        )
        agb.start()
        agb.wait()

        out_ref[0] = ag_ref[...].astype(F32)

    return pl.pallas_call(
        body,
        out_shape=jax.ShapeDtypeStruct((1, SQ, D), F32),
        in_specs=[pl.BlockSpec(memory_space=pltpu.VMEM)] * 5,
        out_specs=pl.BlockSpec(memory_space=pltpu.VMEM),
        scratch_shapes=[
            pltpu.VMEM((SQ, W), BF),
            pltpu.VMEM((SQ // 2, W), BF),
            pltpu.VMEM((SQ // 2, W), BF),
            pltpu.VMEM((CHUNK, W), BF),
            pltpu.VMEM((SQ, D), BF),
            pltpu.SemaphoreType.DMA((4,)),
            pltpu.SemaphoreType.DMA((4,)),
        ],
        compiler_params=pltpu.CompilerParams(
            collective_id=0, vmem_limit_bytes=128 * 1024 * 1024
        ),
    )(x, Wq, K_ext, V_ext, Wo)


# device time: 76411 ns/iter; 2.5744x vs baseline; 1.1043x over previous
import jax
import jax.numpy as jnp
from jax import lax
from jax.experimental import pallas as pl
from jax.experimental.pallas import tpu as pltpu

N_DEV = 4
SQ = 1024
HALF = SQ // 2
SKV_LOC = 1024
HQ = 8
DH = 128
D = HQ * DH
STATS = 128
W = D + STATS
CHUNK = SQ // N_DEV
SCALE = 0.08838834764831843
BLK = 64
BF = jnp.bfloat16
F32 = jnp.float32


def kernel(x, Wq, K_ext, V_ext, Wo):
    def body(x_ref, wq_ref, k_ref, v_ref, wo_ref, out_ref,
             part_ref, comb_ref, recv1_ref, recv2_ref, ag_ref,
             send_sems, recv_sems):
        my = lax.axis_index("i")
        p_x = 3 - my
        p_y = jnp.bitwise_xor(my, 1)

        wq16 = wq_ref[...].astype(BF)
        kfull = k_ref[0]
        vfull = v_ref[0]
        kh16 = [kfull[:, h, :].astype(BF) for h in range(HQ)]
        vh16 = [vfull[:, h, :].astype(BF) for h in range(HQ)]

        cols = lax.broadcasted_iota(jnp.int32, (HALF, SKV_LOC), 1)
        kb = my * (SKV_LOC // BLK) + cols // BLK

        def partial_half(off):
            xh = x_ref[0, pl.ds(off, HALF), :]
            qh = jnp.dot(xh.astype(BF), wq16, preferred_element_type=F32)
            q16 = qh.astype(BF)
            rows = lax.broadcasted_iota(jnp.int32, (HALF, SKV_LOC), 0) + off
            qb = rows // BLK
            mask = (qb == kb) | (kb == 0) | (((qb + kb) % 3) == 0)
            for h in range(HQ):
                s = lax.dot_general(q16[:, h * DH:(h + 1) * DH], kh16[h],
                                    (((1,), (1,)), ((), ())),
                                    preferred_element_type=F32)
                w = jnp.where(mask, jnp.exp(s * SCALE), 0.0)
                lh = jnp.sum(w, axis=1, keepdims=True)
                ctx = jnp.dot(w.astype(BF), vh16[h],
                              preferred_element_type=F32)
                part_ref[pl.ds(off, HALF), h * DH:(h + 1) * DH] = \
                    ctx.astype(BF)
                part_ref[pl.ds(off, HALF), D + h:D + h + 1] = lh.astype(BF)

        send1 = jnp.where(my < 2, HALF, 0)
        keep1 = HALF - send1

        partial_half(send1)

        barrier = pltpu.get_barrier_semaphore()
        for nbr in (p_x, p_y):
            pl.semaphore_signal(barrier, inc=1, device_id=(nbr,),
                                device_id_type=pl.DeviceIdType.MESH)
        pl.semaphore_wait(barrier, 2)

        rs1 = pltpu.make_async_remote_copy(
            src_ref=part_ref.at[pl.ds(send1, HALF)],
            dst_ref=recv1_ref,
            send_sem=send_sems.at[0], recv_sem=recv_sems.at[0],
            device_id=(p_x,), device_id_type=pl.DeviceIdType.MESH,
        )
        rs1.start()

        partial_half(keep1)

        rs1.wait()
        comb = (part_ref[pl.ds(keep1, HALF), :].astype(F32)
                + recv1_ref[...].astype(F32))
        comb_ref[...] = comb.astype(BF)

        keep2 = lax.rem(my, 2) * CHUNK
        send2 = CHUNK - keep2
        rs2 = pltpu.make_async_remote_copy(
            src_ref=comb_ref.at[pl.ds(send2, CHUNK)],
            dst_ref=recv2_ref,
            send_sem=send_sems.at[1], recv_sem=recv_sems.at[1],
            device_id=(p_y,), device_id_type=pl.DeviceIdType.MESH,
        )
        rs2.start()
        rs2.wait()
        final = (comb_ref[pl.ds(keep2, CHUNK), :].astype(F32)
                 + recv2_ref[...].astype(F32))

        outc = jnp.zeros((CHUNK, D), dtype=F32)
        for h in range(HQ):
            ctxn = final[:, h * DH:(h + 1) * DH] / final[:, D + h:D + h + 1]
            outc = outc + jnp.dot(
                ctxn.astype(BF), wo_ref[h * DH:(h + 1) * DH, :].astype(BF),
                preferred_element_type=F32)
        ag_ref[pl.ds(my * CHUNK, CHUNK), :] = outc.astype(BF)

        aga = pltpu.make_async_remote_copy(
            src_ref=ag_ref.at[pl.ds(my * CHUNK, CHUNK)],
            dst_ref=ag_ref.at[pl.ds(my * CHUNK, CHUNK)],
            send_sem=send_sems.at[2], recv_sem=recv_sems.at[2],
            device_id=(p_y,), device_id_type=pl.DeviceIdType.MESH,
        )
        aga.start()
        aga.wait()
        half = (my // 2) * HALF
        agb = pltpu.make_async_remote_copy(
            src_ref=ag_ref.at[pl.ds(half, HALF)],
            dst_ref=ag_ref.at[pl.ds(half, HALF)],
            send_sem=send_sems.at[3], recv_sem=recv_sems.at[3],
            device_id=(p_x,), device_id_type=pl.DeviceIdType.MESH,
        )
        agb.start()
        agb.wait()

        out_ref[0] = ag_ref[...].astype(F32)

    return pl.pallas_call(
        body,
        out_shape=jax.ShapeDtypeStruct((1, SQ, D), F32),
        in_specs=[pl.BlockSpec(memory_space=pltpu.VMEM)] * 5,
        out_specs=pl.BlockSpec(memory_space=pltpu.VMEM),
        scratch_shapes=[
            pltpu.VMEM((SQ, W), BF),
            pltpu.VMEM((HALF, W), BF),
            pltpu.VMEM((HALF, W), BF),
            pltpu.VMEM((CHUNK, W), BF),
            pltpu.VMEM((SQ, D), BF),
            pltpu.SemaphoreType.DMA((4,)),
            pltpu.SemaphoreType.DMA((4,)),
        ],
        compiler_params=pltpu.CompilerParams(
            collective_id=0, vmem_limit_bytes=128 * 1024 * 1024
        ),
    )(x, Wq, K_ext, V_ext, Wo)


# device time: 62287 ns/iter; 3.1582x vs baseline; 1.2268x over previous
import jax
import jax.numpy as jnp
from jax import lax
from jax.experimental import pallas as pl
from jax.experimental.pallas import tpu as pltpu

N_DEV = 4
SQ = 1024
SKV_LOC = 1024
HQ = 8
DH = 128
D = HQ * DH
STATS = 128
W = D + STATS
CHUNK = SQ // N_DEV
SCALE = 0.08838834764831843
BLK = 64
BF = jnp.bfloat16
F32 = jnp.float32


def kernel(x, Wq, K_ext, V_ext, Wo):
    def body(x_ref, wq_ref, k_ref, v_ref, wo_ref, out_ref,
             part_ref, rsrecv_ref, ag_ref,
             rs_send_sems, rs_recv_sems, ag_send_sems, ag_recv_sems):
        my = lax.axis_index("i")

        wq16 = wq_ref[...].astype(BF)
        kfull = k_ref[0]
        vfull = v_ref[0]
        kh16 = [kfull[:, h, :].astype(BF) for h in range(HQ)]
        vh16 = [vfull[:, h, :].astype(BF) for h in range(HQ)]

        cols = lax.broadcasted_iota(jnp.int32, (CHUNK, SKV_LOC), 1)
        kb = my * (SKV_LOC // BLK) + cols // BLK

        def partial_chunk(off):
            xh = x_ref[0, pl.ds(off, CHUNK), :]
            qh = jnp.dot(xh.astype(BF), wq16, preferred_element_type=F32)
            q16 = qh.astype(BF)
            rows = lax.broadcasted_iota(jnp.int32, (CHUNK, SKV_LOC), 0) + off
            qb = rows // BLK
            mask = (qb == kb) | (kb == 0) | (((qb + kb) % 3) == 0)
            for h in range(HQ):
                s = lax.dot_general(q16[:, h * DH:(h + 1) * DH], kh16[h],
                                    (((1,), (1,)), ((), ())),
                                    preferred_element_type=F32)
                w = jnp.where(mask, jnp.exp(s * SCALE), 0.0)
                lh = jnp.sum(w, axis=1, keepdims=True)
                ctx = jnp.dot(w.astype(BF), vh16[h],
                              preferred_element_type=F32)
                part_ref[pl.ds(off, CHUNK), h * DH:(h + 1) * DH] = \
                    ctx.astype(BF)
                part_ref[pl.ds(off, CHUNK), D + h:D + h + 1] = lh.astype(BF)

        barrier = pltpu.get_barrier_semaphore()
        for k in range(1, N_DEV):
            peer = lax.rem(my + k, N_DEV)
            pl.semaphore_signal(barrier, inc=1, device_id=(peer,),
                                device_id_type=pl.DeviceIdType.MESH)
        pl.semaphore_wait(barrier, N_DEV - 1)

        rs_rdmas = []
        for k in range(N_DEV - 1):
            t = lax.rem(my + 1 + k, N_DEV)
            partial_chunk(t * CHUNK)
            rdma = pltpu.make_async_remote_copy(
                src_ref=part_ref.at[pl.ds(t * CHUNK, CHUNK)],
                dst_ref=rsrecv_ref.at[k],
                send_sem=rs_send_sems.at[k], recv_sem=rs_recv_sems.at[k],
                device_id=(t,), device_id_type=pl.DeviceIdType.MESH,
            )
            rdma.start()
            rs_rdmas.append(rdma)

        partial_chunk(my * CHUNK)

        for rdma in rs_rdmas:
            rdma.wait()

        final = (part_ref[pl.ds(my * CHUNK, CHUNK), :].astype(F32)
                 + rsrecv_ref[0].astype(F32)
                 + rsrecv_ref[1].astype(F32)
                 + rsrecv_ref[2].astype(F32))

        outc = jnp.zeros((CHUNK, D), dtype=F32)
        for h in range(HQ):
            ctxn = final[:, h * DH:(h + 1) * DH] / final[:, D + h:D + h + 1]
            outc = outc + jnp.dot(
                ctxn.astype(BF), wo_ref[h * DH:(h + 1) * DH, :].astype(BF),
                preferred_element_type=F32)
        ag_ref[pl.ds(my * CHUNK, CHUNK), :] = outc.astype(BF)

        ag_rdmas = []
        for k in range(N_DEV - 1):
            t = lax.rem(my + 1 + k, N_DEV)
            rdma = pltpu.make_async_remote_copy(
                src_ref=ag_ref.at[pl.ds(my * CHUNK, CHUNK)],
                dst_ref=ag_ref.at[pl.ds(my * CHUNK, CHUNK)],
                send_sem=ag_send_sems.at[k], recv_sem=ag_recv_sems.at[k],
                device_id=(t,), device_id_type=pl.DeviceIdType.MESH,
            )
            rdma.start()
            ag_rdmas.append(rdma)
        for rdma in ag_rdmas:
            rdma.wait()

        out_ref[0] = ag_ref[...].astype(F32)

    return pl.pallas_call(
        body,
        out_shape=jax.ShapeDtypeStruct((1, SQ, D), F32),
        in_specs=[pl.BlockSpec(memory_space=pltpu.VMEM)] * 5,
        out_specs=pl.BlockSpec(memory_space=pltpu.VMEM),
        scratch_shapes=[
            pltpu.VMEM((SQ, W), BF),
            pltpu.VMEM((N_DEV - 1, CHUNK, W), BF),
            pltpu.VMEM((SQ, D), BF),
            pltpu.SemaphoreType.DMA((N_DEV - 1,)),
            pltpu.SemaphoreType.DMA((N_DEV - 1,)),
            pltpu.SemaphoreType.DMA((N_DEV - 1,)),
            pltpu.SemaphoreType.DMA((N_DEV - 1,)),
        ],
        compiler_params=pltpu.CompilerParams(
            collective_id=0, vmem_limit_bytes=128 * 1024 * 1024
        ),
    )(x, Wq, K_ext, V_ext, Wo)


# device time: 60503 ns/iter; 3.2513x vs baseline; 1.0295x over previous
import jax
import jax.numpy as jnp
from jax import lax
from jax.experimental import pallas as pl
from jax.experimental.pallas import tpu as pltpu

N_DEV = 4
SQ = 1024
SKV_LOC = 1024
HQ = 8
DH = 128
D = HQ * DH
STATS = 128
W = D + STATS
CHUNK = SQ // N_DEV
SCALE = 0.08838834764831843
BLK = 64
BF = jnp.bfloat16
F32 = jnp.float32


def kernel(x, Wq, K_ext, V_ext, Wo):
    def body(x_ref, wq_ref, k_ref, v_ref, wo_ref, out_ref,
             part_ref, rsrecv_ref, ag_ref,
             rs_send_sems, rs_recv_sems, ag_send_sems, ag_recv_sems):
        my = lax.axis_index("i")

        wq16 = wq_ref[...].astype(BF)
        kfull = k_ref[0]
        vfull = v_ref[0]
        kh16 = [kfull[:, h, :].astype(BF) for h in range(HQ)]
        vh16 = [vfull[:, h, :].astype(BF) for h in range(HQ)]

        cols = lax.broadcasted_iota(jnp.int32, (CHUNK, SKV_LOC), 1)
        kb = my * (SKV_LOC // BLK) + cols // BLK

        def partial_chunk(off):
            xh = x_ref[0, pl.ds(off, CHUNK), :]
            qh = jnp.dot(xh.astype(BF), wq16, preferred_element_type=F32)
            q16 = qh.astype(BF)
            rows = lax.broadcasted_iota(jnp.int32, (CHUNK, SKV_LOC), 0) + off
            qb = rows // BLK
            mask = (qb == kb) | (kb == 0) | (((qb + kb) % 3) == 0)
            for h in range(HQ):
                s = lax.dot_general(q16[:, h * DH:(h + 1) * DH], kh16[h],
                                    (((1,), (1,)), ((), ())),
                                    preferred_element_type=F32)
                w = jnp.where(mask, jnp.exp(s * SCALE), 0.0)
                lh = jnp.sum(w, axis=1, keepdims=True)
                ctx = jnp.dot(w.astype(BF), vh16[h],
                              preferred_element_type=F32)
                part_ref[pl.ds(off, CHUNK), h * DH:(h + 1) * DH] = \
                    ctx.astype(BF)
                part_ref[pl.ds(off, CHUNK), D + h:D + h + 1] = lh.astype(BF)

        barrier = pltpu.get_barrier_semaphore()
        for k in range(1, N_DEV):
            peer = lax.rem(my + k, N_DEV)
            pl.semaphore_signal(barrier, inc=1, device_id=(peer,),
                                device_id_type=pl.DeviceIdType.MESH)

        rs_rdmas = []
        for k in range(N_DEV - 1):
            t = lax.rem(my + 1 + k, N_DEV)
            partial_chunk(t * CHUNK)
            if k == 0:
                pl.semaphore_wait(barrier, N_DEV - 1)
            rdma = pltpu.make_async_remote_copy(
                src_ref=part_ref.at[pl.ds(t * CHUNK, CHUNK)],
                dst_ref=rsrecv_ref.at[k],
                send_sem=rs_send_sems.at[k], recv_sem=rs_recv_sems.at[k],
                device_id=(t,), device_id_type=pl.DeviceIdType.MESH,
            )
            rdma.start()
            rs_rdmas.append(rdma)

        partial_chunk(my * CHUNK)

        for rdma in rs_rdmas:
            rdma.wait()

        final = (part_ref[pl.ds(my * CHUNK, CHUNK), :].astype(F32)
                 + rsrecv_ref[0].astype(F32)
                 + rsrecv_ref[1].astype(F32)
                 + rsrecv_ref[2].astype(F32))

        ctxn = jnp.concatenate(
            [final[:, h * DH:(h + 1) * DH] / final[:, D + h:D + h + 1]
             for h in range(HQ)], axis=1)
        outc = jnp.dot(ctxn.astype(BF), wo_ref[...].astype(BF),
                       preferred_element_type=F32)
        ag_ref[pl.ds(my * CHUNK, CHUNK), :] = outc.astype(BF)
        out_ref[0, pl.ds(my * CHUNK, CHUNK), :] = outc

        ag_rdmas = []
        for k in range(N_DEV - 1):
            t = lax.rem(my + 1 + k, N_DEV)
            rdma = pltpu.make_async_remote_copy(
                src_ref=ag_ref.at[pl.ds(my * CHUNK, CHUNK)],
                dst_ref=ag_ref.at[pl.ds(my * CHUNK, CHUNK)],
                send_sem=ag_send_sems.at[k], recv_sem=ag_recv_sems.at[k],
                device_id=(t,), device_id_type=pl.DeviceIdType.MESH,
            )
            rdma.start()
            ag_rdmas.append(rdma)
        for rdma in ag_rdmas:
            rdma.wait()
        for k in range(N_DEV - 1):
            t = lax.rem(my + 1 + k, N_DEV)
            out_ref[0, pl.ds(t * CHUNK, CHUNK), :] = \
                ag_ref[pl.ds(t * CHUNK, CHUNK), :].astype(F32)

    return pl.pallas_call(
        body,
        out_shape=jax.ShapeDtypeStruct((1, SQ, D), F32),
        in_specs=[pl.BlockSpec(memory_space=pltpu.VMEM)] * 5,
        out_specs=pl.BlockSpec(memory_space=pltpu.VMEM),
        scratch_shapes=[
            pltpu.VMEM((SQ, W), BF),
            pltpu.VMEM((N_DEV - 1, CHUNK, W), BF),
            pltpu.VMEM((SQ, D), BF),
            pltpu.SemaphoreType.DMA((N_DEV - 1,)),
            pltpu.SemaphoreType.DMA((N_DEV - 1,)),
            pltpu.SemaphoreType.DMA((N_DEV - 1,)),
            pltpu.SemaphoreType.DMA((N_DEV - 1,)),
        ],
        compiler_params=pltpu.CompilerParams(
            collective_id=0, vmem_limit_bytes=128 * 1024 * 1024
        ),
    )(x, Wq, K_ext, V_ext, Wo)


# device time: 60298 ns/iter; 3.2624x vs baseline; 1.0034x over previous
import jax
import jax.numpy as jnp
from jax import lax
from jax.experimental import pallas as pl
from jax.experimental.pallas import tpu as pltpu

N_DEV = 4
SQ = 1024
SKV_LOC = 1024
HQ = 8
DH = 128
D = HQ * DH
STATS = 128
W = D + STATS
CHUNK = SQ // N_DEV
SCALE = 0.08838834764831843
BLK = 64
BF = jnp.bfloat16
F32 = jnp.float32


def kernel(x, Wq, K_ext, V_ext, Wo):
    def body(x_ref, wq_ref, k_ref, v_ref, wo_ref, out_ref,
             part_ref, rsrecv_ref, ag_ref,
             rs_send_sems, rs_recv_sems, ag_send_sems, ag_recv_sems):
        my = lax.axis_index("i")

        wq16 = wq_ref[...].astype(BF)
        kfull = k_ref[0]
        vfull = v_ref[0]
        kh16 = [kfull[:, h, :].astype(BF) for h in range(HQ)]
        vh16 = [vfull[:, h, :].astype(BF) for h in range(HQ)]

        cols = lax.broadcasted_iota(jnp.int32, (CHUNK, SKV_LOC), 1)
        kb = my * (SKV_LOC // BLK) + cols // BLK

        def partial_chunk(off):
            xh = x_ref[0, pl.ds(off, CHUNK), :]
            qh = jnp.dot(xh.astype(BF), wq16, preferred_element_type=F32)
            q16 = qh.astype(BF)
            rows = lax.broadcasted_iota(jnp.int32, (CHUNK, SKV_LOC), 0) + off
            qb = rows // BLK
            mask = (qb == kb) | (kb == 0) | (((qb + kb) % 3) == 0)
            for h in range(HQ):
                s = lax.dot_general(q16[:, h * DH:(h + 1) * DH], kh16[h],
                                    (((1,), (1,)), ((), ())),
                                    preferred_element_type=F32)
                w = jnp.where(mask, jnp.exp(s * SCALE), 0.0)
                lh = jnp.sum(w, axis=1, keepdims=True)
                ctx = jnp.dot(w.astype(BF), vh16[h],
                              preferred_element_type=F32)
                part_ref[pl.ds(off, CHUNK), h * DH:(h + 1) * DH] = \
                    ctx.astype(BF)
                part_ref[pl.ds(off, CHUNK), D + h:D + h + 1] = lh.astype(BF)

        barrier = pltpu.get_barrier_semaphore()
        for k in range(1, N_DEV):
            peer = lax.rem(my + k, N_DEV)
            pl.semaphore_signal(barrier, inc=1, device_id=(peer,),
                                device_id_type=pl.DeviceIdType.MESH)

        rs_rdmas = []
        for k in range(N_DEV - 1):
            t = lax.rem(my + 1 + k, N_DEV)
            partial_chunk(t * CHUNK)
            if k == 0:
                pl.semaphore_wait(barrier, N_DEV - 1)
            rdma = pltpu.make_async_remote_copy(
                src_ref=part_ref.at[pl.ds(t * CHUNK, CHUNK)],
                dst_ref=rsrecv_ref.at[k],
                send_sem=rs_send_sems.at[k], recv_sem=rs_recv_sems.at[k],
                device_id=(t,), device_id_type=pl.DeviceIdType.MESH,
            )
            rdma.start()
            rs_rdmas.append(rdma)

        partial_chunk(my * CHUNK)

        final = part_ref[pl.ds(my * CHUNK, CHUNK), :].astype(F32)
        for k in range(N_DEV - 1):
            rs_rdmas[k].wait()
            final = final + rsrecv_ref[k].astype(F32)

        ctxn = jnp.concatenate(
            [final[:, h * DH:(h + 1) * DH] / final[:, D + h:D + h + 1]
             for h in range(HQ)], axis=1)
        ctx16 = ctxn.astype(BF)
        wo16 = wo_ref[...].astype(BF)

        ag_rdmas = []
        for half in range(2):
            csl = pl.ds(half * (D // 2), D // 2)
            outc = jnp.dot(ctx16, wo16[:, half * (D // 2):(half + 1) * (D // 2)],
                           preferred_element_type=F32)
            ag_ref[pl.ds(my * CHUNK, CHUNK), csl] = outc.astype(BF)
            out_ref[0, pl.ds(my * CHUNK, CHUNK), csl] = outc
            for k in range(N_DEV - 1):
                t = lax.rem(my + 1 + k, N_DEV)
                idx = half * (N_DEV - 1) + k
                rdma = pltpu.make_async_remote_copy(
                    src_ref=ag_ref.at[pl.ds(my * CHUNK, CHUNK), csl],
                    dst_ref=ag_ref.at[pl.ds(my * CHUNK, CHUNK), csl],
                    send_sem=ag_send_sems.at[idx], recv_sem=ag_recv_sems.at[idx],
                    device_id=(t,), device_id_type=pl.DeviceIdType.MESH,
                )
                rdma.start()
                ag_rdmas.append(rdma)
        for rdma in ag_rdmas:
            rdma.wait()
        for k in range(N_DEV - 1):
            t = lax.rem(my + 1 + k, N_DEV)
            out_ref[0, pl.ds(t * CHUNK, CHUNK), :] = \
                ag_ref[pl.ds(t * CHUNK, CHUNK), :].astype(F32)

    return pl.pallas_call(
        body,
        out_shape=jax.ShapeDtypeStruct((1, SQ, D), F32),
        in_specs=[pl.BlockSpec(memory_space=pltpu.VMEM)] * 5,
        out_specs=pl.BlockSpec(memory_space=pltpu.VMEM),
        scratch_shapes=[
            pltpu.VMEM((SQ, W), BF),
            pltpu.VMEM((N_DEV - 1, CHUNK, W), BF),
            pltpu.VMEM((SQ, D), BF),
            pltpu.SemaphoreType.DMA((N_DEV - 1,)),
            pltpu.SemaphoreType.DMA((N_DEV - 1,)),
            pltpu.SemaphoreType.DMA((2 * (N_DEV - 1),)),
            pltpu.SemaphoreType.DMA((2 * (N_DEV - 1),)),
        ],
        compiler_params=pltpu.CompilerParams(
            collective_id=0, vmem_limit_bytes=128 * 1024 * 1024
        ),
    )(x, Wq, K_ext, V_ext, Wo)
